# Initial kernel scaffold; baseline (speedup 1.0000x reference)
import jax
import jax.numpy as jnp
from jax.experimental import pallas as pl


def kernel(emb1, emb2, targets, keep_rate):
    raise NotImplementedError("write your pallas kernel here")


# speedup vs baseline: 26.5224x; 26.5224x over previous
import functools

import jax
import jax.numpy as jnp
import numpy as np
from jax.experimental import pallas as pl

_B = 256
_ABLK = 8
_GRID = _B // _ABLK
_TRUNC = -4096
_BIG = 0x7FFFFFFF

_R1T = np.array(
    [((j << 28) | 0x0FFFFFFF) ^ 0x80000000 for j in range(16)], dtype=np.uint32
).view(np.int32)


def _sortkey(x):
    i = jax.lax.bitcast_convert_type(x, jnp.int32)
    return jnp.where(i >= 0, i, i ^ 0x7FFFFFFF)


def _softplus(x):
    return jnp.maximum(x, 0.0) + jnp.log1p(jnp.exp(-jnp.abs(x)))


def _onehot_cell(r, c):
    ri = jax.lax.broadcasted_iota(jnp.int32, (8, 128), 0)
    ci = jax.lax.broadcasted_iota(jnp.int32, (8, 128), 1)
    return (ri == r) & (ci == c)


def _cell_f32(scalar, r, c):
    return jnp.where(_onehot_cell(r, c), scalar, 0.0)


def _extract_i32(arr, r, c):
    return jnp.sum(jnp.where(_onehot_cell(r, c), arr, 0))


def _extract_f32(arr, r, c):
    return jnp.sum(jnp.where(_onehot_cell(r, c), arr, 0.0))


def _cumsum_last(x, n):
    sh = 1
    while sh < n:
        pad = jnp.zeros_like(jax.lax.slice_in_dim(x, 0, sh, axis=x.ndim - 1))
        x = x + jnp.concatenate(
            [pad, jax.lax.slice_in_dim(x, 0, x.shape[-1] - sh, axis=x.ndim - 1)],
            axis=x.ndim - 1,
        )
        sh *= 2
    return x


def _cumsum_axis(x, axis, n):
    sh = 1
    while sh < n:
        pad = jnp.zeros_like(jax.lax.slice_in_dim(x, 0, sh, axis=axis))
        x = x + jnp.concatenate(
            [pad, jax.lax.slice_in_dim(x, 0, x.shape[axis] - sh, axis=axis)],
            axis=axis,
        )
        sh *= 2
    return x


def _dist_kernel(e1_ref, e2_ref, d1_ref, d2_ref):
    eye = (
        jax.lax.broadcasted_iota(jnp.int32, (_B, _B), 0)
        == jax.lax.broadcasted_iota(jnp.int32, (_B, _B), 1)
    ).astype(jnp.float32)
    for e_ref, d_ref in ((e1_ref, d1_ref), (e2_ref, d2_ref)):
        e = e_ref[...]
        g = jax.lax.dot_general(
            e, e, (((1,), (1,)), ((), ())), preferred_element_type=jnp.float32
        )
        ncol = jnp.sum(e * e, axis=1, keepdims=True)
        nrow = jnp.sum(g * eye, axis=0, keepdims=True)
        d_ref[...] = ncol + nrow - 2.0 * g


def _masked_keys(d_ref, pv_ref, nv_ref):
    da = d_ref[...]
    diff = da[:, :, None] - da[:, None, :]
    pvb = pv_ref[...]
    nvb = nv_ref[...]
    vmf = pvb[:, :, None] * nvb[:, None, :]
    key = jnp.where(vmf > 0, _sortkey(diff), _BIG)
    return diff, vmf, key


def _count_kernel(thr_ref, d1_ref, d2_ref, pv_ref, nv_ref, out_ref, *, first_round):
    a = pl.program_id(0)

    @pl.when(a == 0)
    def _init():
        out_ref[...] = jnp.zeros((8, 128), jnp.float32)

    contrib = jnp.zeros((8, 128), jnp.float32)
    for t_idx, d_ref in ((0, d1_ref), (1, d2_ref)):
        _, vmf, key = _masked_keys(d_ref, pv_ref, nv_ref)
        if first_round and t_idx == 0:
            contrib = contrib + _cell_f32(jnp.sum(vmf), 0, 0)
        for j in range(16):
            if first_round:
                t = int(_R1T[j])
            else:
                t = _extract_i32(thr_ref[...], t_idx, j)
            cnt = jnp.sum(jnp.where(key <= t, 1.0, 0.0))
            contrib = contrib + _cell_f32(cnt, t_idx + 1, j)
    out_ref[...] = out_ref[...] + contrib


def _final_kernel(sc_ref, d1_ref, d2_ref, pv_ref, nv_ref, out_ref):
    a = pl.program_id(0)

    @pl.when(a == 0)
    def _init():
        out_ref[...] = jnp.zeros((8, 128), jnp.float32)

    sc = sc_ref[...]
    v1 = _extract_i32(sc, 0, 0)
    v2 = _extract_i32(sc, 0, 1)
    r1 = _extract_i32(sc, 0, 2).astype(jnp.float32)
    r2 = _extract_i32(sc, 0, 3).astype(jnp.float32)

    diff1, vmf, key1 = _masked_keys(d1_ref, pv_ref, nv_ref)
    diff2, _, key2 = _masked_keys(d2_ref, pv_ref, nv_ref)
    loss1 = _softplus(diff1)
    loss2 = _softplus(diff2)

    old = out_ref[...]
    run2 = _extract_f32(old, 1, 0)
    run1 = _extract_f32(old, 1, 1)

    contrib = jnp.zeros((8, 128), jnp.float32)
    contrib = contrib + _cell_f32(jnp.sum(vmf * loss1), 0, 2)
    contrib = contrib + _cell_f32(jnp.sum(vmf * loss2), 0, 3)

    for (sel_key, vstar, run, rneed, other_loss, col, runcol) in (
        (key2, v2, run2, r2, loss1, 0, 0),
        (key1, v1, run1, r1, loss2, 1, 1),
    ):
        tkey = sel_key & _TRUNC
        lt = tkey < vstar
        tie = (tkey == vstar).astype(jnp.float32)
        s_lt = jnp.sum(jnp.where(lt, other_loss, 0.0))
        cs_n = _cumsum_last(tie, _B)
        row_tot = jax.lax.slice_in_dim(cs_n, _B - 1, _B, axis=2)
        incl_p = _cumsum_axis(row_tot, 1, _B)
        excl_p = incl_p - row_tot
        a_tot = jax.lax.slice_in_dim(incl_p, _B - 1, _B, axis=1)
        incl_a = _cumsum_axis(a_tot, 0, _ABLK)
        excl_a = incl_a - a_tot
        local_excl = (cs_n - tie) + excl_p + excl_a
        take = (tie > 0) & (run + local_excl < rneed)
        s_tie = jnp.sum(jnp.where(take, other_loss, 0.0))
        contrib = contrib + _cell_f32(s_lt + s_tie, 0, col)
        contrib = contrib + _cell_f32(jnp.sum(tie), 1, runcol)

    out_ref[...] = old + contrib


def _sweep_call(kern, d1, d2, pv, nv, carrier):
    return pl.pallas_call(
        kern,
        grid=(_GRID,),
        in_specs=[
            pl.BlockSpec((8, 128), lambda a: (0, 0)),
            pl.BlockSpec((_ABLK, _B), lambda a: (a, 0)),
            pl.BlockSpec((_ABLK, _B), lambda a: (a, 0)),
            pl.BlockSpec((_ABLK, _B), lambda a: (a, 0)),
            pl.BlockSpec((_ABLK, _B), lambda a: (a, 0)),
        ],
        out_specs=pl.BlockSpec((8, 128), lambda a: (0, 0)),
        out_shape=jax.ShapeDtypeStruct((8, 128), jnp.float32),
    )(carrier, d1, d2, pv, nv)


def kernel(emb1, emb2, targets, keep_rate):
    same = targets[:, None] == targets[None, :]
    idx = jnp.arange(_B)
    pv = (same & (idx[:, None] < idx[None, :])).astype(jnp.float32)
    nv = (~same).astype(jnp.float32)

    d1, d2 = pl.pallas_call(
        _dist_kernel,
        out_shape=(
            jax.ShapeDtypeStruct((_B, _B), jnp.float32),
            jax.ShapeDtypeStruct((_B, _B), jnp.float32),
        ),
    )(emb1, emb2)

    zero_c = jnp.zeros((8, 128), jnp.int32)
    o2 = _sweep_call(
        functools.partial(_count_kernel, first_round=True), d1, d2, pv, nv, zero_c
    )
    num_valid_f = o2[0, 0]
    num_valid = num_valid_f.astype(jnp.int32)
    num_keep = jnp.floor(keep_rate * num_valid).astype(jnp.int32)
    kf = num_keep.astype(jnp.float32)

    jarr = jnp.arange(16, dtype=jnp.uint32)
    bit31 = jnp.uint32(0x80000000)

    def pick(C, prefix, cbelow, shift):
        ge = C >= kf
        jstar = jnp.argmax(ge).astype(jnp.uint32)
        cb_new = jnp.where(
            jstar > 0, C[jnp.maximum(jstar.astype(jnp.int32) - 1, 0)], cbelow
        )
        return prefix | (jstar << shift), cb_new

    p1, cb1 = pick(o2[1, :16], jnp.uint32(0), jnp.float32(0.0), 28)
    p2, cb2 = pick(o2[2, :16], jnp.uint32(0), jnp.float32(0.0), 28)

    for shift in (24, 20, 16, 12):
        low = jnp.uint32((1 << shift) - 1)
        t1 = jax.lax.bitcast_convert_type(
            (p1 | (jarr << shift) | low) ^ bit31, jnp.int32
        )
        t2 = jax.lax.bitcast_convert_type(
            (p2 | (jarr << shift) | low) ^ bit31, jnp.int32
        )
        carrier = zero_c.at[0, :16].set(t1).at[1, :16].set(t2)
        o3 = _sweep_call(
            functools.partial(_count_kernel, first_round=False), d1, d2, pv, nv, carrier
        )
        p1, cb1 = pick(o3[1, :16], p1, cb1, shift)
        p2, cb2 = pick(o3[2, :16], p2, cb2, shift)

    v1s = jax.lax.bitcast_convert_type(p1 ^ bit31, jnp.int32)
    v2s = jax.lax.bitcast_convert_type(p2 ^ bit31, jnp.int32)
    r1 = num_keep - cb1.astype(jnp.int32)
    r2 = num_keep - cb2.astype(jnp.int32)

    sc = (
        zero_c.at[0, 0]
        .set(v1s)
        .at[0, 1]
        .set(v2s)
        .at[0, 2]
        .set(r1)
        .at[0, 3]
        .set(r2)
    )
    o4 = _sweep_call(_final_kernel, d1, d2, pv, nv, sc)

    upd1, upd2 = o4[0, 0], o4[0, 1]
    l1s, l2s = o4[0, 2], o4[0, 3]
    return (upd1 / num_keep, upd2 / num_keep, l1s / num_valid, l2s / num_valid)


# speedup vs baseline: 32.5826x; 1.2285x over previous
import functools

import jax
import jax.numpy as jnp
import numpy as np
from jax.experimental import pallas as pl

_B = 256
_ABLK = 8
_GRID = _B // _ABLK
_TRUNC = -65536
_BIG = 0x7FFFFFFF

_R1T = np.array(
    [((j << 28) | 0x0FFFFFFF) ^ 0x80000000 for j in range(16)], dtype=np.uint32
).view(np.int32)


def _sortkey(x):
    i = jax.lax.bitcast_convert_type(x, jnp.int32)
    return jnp.where(i >= 0, i, i ^ 0x7FFFFFFF)


def _softplus(x):
    return jnp.maximum(x, 0.0) + jnp.log1p(jnp.exp(-jnp.abs(x)))


def _onehot_cell(r, c):
    ri = jax.lax.broadcasted_iota(jnp.int32, (8, 128), 0)
    ci = jax.lax.broadcasted_iota(jnp.int32, (8, 128), 1)
    return (ri == r) & (ci == c)


def _cell_f32(scalar, r, c):
    return jnp.where(_onehot_cell(r, c), scalar, 0.0)


def _extract_i32(arr, r, c):
    return jnp.sum(jnp.where(_onehot_cell(r, c), arr, 0))


def _extract_f32(arr, r, c):
    return jnp.sum(jnp.where(_onehot_cell(r, c), arr, 0.0))


def _cumsum_last(x, n):
    sh = 1
    while sh < n:
        pad = jnp.zeros_like(jax.lax.slice_in_dim(x, 0, sh, axis=x.ndim - 1))
        x = x + jnp.concatenate(
            [pad, jax.lax.slice_in_dim(x, 0, x.shape[-1] - sh, axis=x.ndim - 1)],
            axis=x.ndim - 1,
        )
        sh *= 2
    return x


def _cumsum_axis(x, axis, n):
    sh = 1
    while sh < n:
        pad = jnp.zeros_like(jax.lax.slice_in_dim(x, 0, sh, axis=axis))
        x = x + jnp.concatenate(
            [pad, jax.lax.slice_in_dim(x, 0, x.shape[axis] - sh, axis=axis)],
            axis=axis,
        )
        sh *= 2
    return x


def _dist_kernel(e1_ref, e2_ref, d1_ref, d2_ref):
    eye = (
        jax.lax.broadcasted_iota(jnp.int32, (_B, _B), 0)
        == jax.lax.broadcasted_iota(jnp.int32, (_B, _B), 1)
    ).astype(jnp.float32)
    for e_ref, d_ref in ((e1_ref, d1_ref), (e2_ref, d2_ref)):
        e = e_ref[...]
        g = jax.lax.dot_general(
            e, e, (((1,), (1,)), ((), ())), preferred_element_type=jnp.float32
        )
        ncol = jnp.sum(e * e, axis=1, keepdims=True)
        nrow = jnp.sum(g * eye, axis=0, keepdims=True)
        d_ref[...] = ncol + nrow - 2.0 * g


def _valid_mask(pv_ref, nv_ref):
    pvb = pv_ref[...]
    nvb = nv_ref[...]
    return pvb[:, :, None] * nvb[:, None, :]


def _masked_keys(d_ref, vmb):
    da = d_ref[...]
    diff = da[:, :, None] - da[:, None, :]
    key = jnp.where(vmb, _sortkey(diff), _BIG)
    return diff, key


def _count_kernel(thr_ref, d1_ref, d2_ref, pv_ref, nv_ref, out_ref, *, first_round):
    a = pl.program_id(0)

    @pl.when(a == 0)
    def _init():
        out_ref[...] = jnp.zeros((8, 128), jnp.float32)

    vmf = _valid_mask(pv_ref, nv_ref)
    vmb = vmf > 0
    contrib = jnp.zeros((8, 128), jnp.float32)
    for t_idx, d_ref in ((0, d1_ref), (1, d2_ref)):
        _, key = _masked_keys(d_ref, vmb)
        if first_round and t_idx == 0:
            contrib = contrib + _cell_f32(jnp.sum(vmf), 0, 0)
        for j in range(16):
            if first_round:
                t = int(_R1T[j])
            else:
                t = _extract_i32(thr_ref[...], t_idx, j)
            cnt = jnp.sum(jnp.where(key <= t, 1.0, 0.0))
            contrib = contrib + _cell_f32(cnt, t_idx + 1, j)
    out_ref[...] = out_ref[...] + contrib


def _final_kernel(sc_ref, d1_ref, d2_ref, pv_ref, nv_ref, out_ref):
    a = pl.program_id(0)

    @pl.when(a == 0)
    def _init():
        out_ref[...] = jnp.zeros((8, 128), jnp.float32)

    sc = sc_ref[...]
    v1 = _extract_i32(sc, 0, 0)
    v2 = _extract_i32(sc, 0, 1)
    r1 = _extract_i32(sc, 0, 2).astype(jnp.float32)
    r2 = _extract_i32(sc, 0, 3).astype(jnp.float32)

    vmf = _valid_mask(pv_ref, nv_ref)
    vmb = vmf > 0
    diff1, key1 = _masked_keys(d1_ref, vmb)
    diff2, key2 = _masked_keys(d2_ref, vmb)
    loss1 = _softplus(diff1)
    loss2 = _softplus(diff2)

    old = out_ref[...]
    run2 = _extract_f32(old, 1, 0)
    run1 = _extract_f32(old, 1, 1)

    contrib = jnp.zeros((8, 128), jnp.float32)
    contrib = contrib + _cell_f32(jnp.sum(vmf * loss1), 0, 2)
    contrib = contrib + _cell_f32(jnp.sum(vmf * loss2), 0, 3)

    for (sel_key, vstar, run, rneed, other_loss, col, runcol) in (
        (key2, v2, run2, r2, loss1, 0, 0),
        (key1, v1, run1, r1, loss2, 1, 1),
    ):
        tkey = sel_key & _TRUNC
        lt = tkey < vstar
        tie = (tkey == vstar).astype(jnp.float32)
        s_lt = jnp.sum(jnp.where(lt, other_loss, 0.0))
        cs_n = _cumsum_last(tie, _B)
        row_tot = jax.lax.slice_in_dim(cs_n, _B - 1, _B, axis=2)
        incl_p = _cumsum_axis(row_tot, 1, _B)
        excl_p = incl_p - row_tot
        a_tot = jax.lax.slice_in_dim(incl_p, _B - 1, _B, axis=1)
        incl_a = _cumsum_axis(a_tot, 0, _ABLK)
        excl_a = incl_a - a_tot
        local_excl = (cs_n - tie) + excl_p + excl_a
        take = (tie > 0) & (run + local_excl < rneed)
        s_tie = jnp.sum(jnp.where(take, other_loss, 0.0))
        contrib = contrib + _cell_f32(s_lt + s_tie, 0, col)
        contrib = contrib + _cell_f32(jnp.sum(tie), 1, runcol)

    out_ref[...] = old + contrib


def _sweep_call(kern, d1, d2, pv, nv, carrier):
    return pl.pallas_call(
        kern,
        grid=(_GRID,),
        in_specs=[
            pl.BlockSpec((8, 128), lambda a: (0, 0)),
            pl.BlockSpec((_ABLK, _B), lambda a: (a, 0)),
            pl.BlockSpec((_ABLK, _B), lambda a: (a, 0)),
            pl.BlockSpec((_ABLK, _B), lambda a: (a, 0)),
            pl.BlockSpec((_ABLK, _B), lambda a: (a, 0)),
        ],
        out_specs=pl.BlockSpec((8, 128), lambda a: (0, 0)),
        out_shape=jax.ShapeDtypeStruct((8, 128), jnp.float32),
    )(carrier, d1, d2, pv, nv)


def kernel(emb1, emb2, targets, keep_rate):
    same = targets[:, None] == targets[None, :]
    idx = jnp.arange(_B)
    pv = (same & (idx[:, None] < idx[None, :])).astype(jnp.float32)
    nv = (~same).astype(jnp.float32)

    d1, d2 = pl.pallas_call(
        _dist_kernel,
        out_shape=(
            jax.ShapeDtypeStruct((_B, _B), jnp.float32),
            jax.ShapeDtypeStruct((_B, _B), jnp.float32),
        ),
    )(emb1, emb2)

    zero_c = jnp.zeros((8, 128), jnp.int32)
    o2 = _sweep_call(
        functools.partial(_count_kernel, first_round=True), d1, d2, pv, nv, zero_c
    )
    num_valid_f = o2[0, 0]
    num_valid = num_valid_f.astype(jnp.int32)
    num_keep = jnp.floor(keep_rate * num_valid).astype(jnp.int32)
    kf = num_keep.astype(jnp.float32)

    jarr = jnp.arange(16, dtype=jnp.uint32)
    bit31 = jnp.uint32(0x80000000)

    def pick(C, prefix, cbelow, shift):
        ge = C >= kf
        jstar = jnp.argmax(ge).astype(jnp.uint32)
        cb_new = jnp.where(
            jstar > 0, C[jnp.maximum(jstar.astype(jnp.int32) - 1, 0)], cbelow
        )
        return prefix | (jstar << shift), cb_new

    p1, cb1 = pick(o2[1, :16], jnp.uint32(0), jnp.float32(0.0), 28)
    p2, cb2 = pick(o2[2, :16], jnp.uint32(0), jnp.float32(0.0), 28)

    for shift in (24, 20, 16):
        low = jnp.uint32((1 << shift) - 1)
        t1 = jax.lax.bitcast_convert_type(
            (p1 | (jarr << shift) | low) ^ bit31, jnp.int32
        )
        t2 = jax.lax.bitcast_convert_type(
            (p2 | (jarr << shift) | low) ^ bit31, jnp.int32
        )
        carrier = zero_c.at[0, :16].set(t1).at[1, :16].set(t2)
        o3 = _sweep_call(
            functools.partial(_count_kernel, first_round=False), d1, d2, pv, nv, carrier
        )
        p1, cb1 = pick(o3[1, :16], p1, cb1, shift)
        p2, cb2 = pick(o3[2, :16], p2, cb2, shift)

    v1s = jax.lax.bitcast_convert_type(p1 ^ bit31, jnp.int32)
    v2s = jax.lax.bitcast_convert_type(p2 ^ bit31, jnp.int32)
    r1 = num_keep - cb1.astype(jnp.int32)
    r2 = num_keep - cb2.astype(jnp.int32)

    sc = (
        zero_c.at[0, 0]
        .set(v1s)
        .at[0, 1]
        .set(v2s)
        .at[0, 2]
        .set(r1)
        .at[0, 3]
        .set(r2)
    )
    o4 = _sweep_call(_final_kernel, d1, d2, pv, nv, sc)

    upd1, upd2 = o4[0, 0], o4[0, 1]
    l1s, l2s = o4[0, 2], o4[0, 3]
    return (upd1 / num_keep, upd2 / num_keep, l1s / num_valid, l2s / num_valid)


# speedup vs baseline: 42.2463x; 1.2966x over previous
import functools

import jax
import jax.numpy as jnp
import numpy as np
from jax.experimental import pallas as pl

_B = 256
_ABLK = 8
_GRID = _B // _ABLK
_TRUNC = -1048576
_BIG = 0x7FFFFFFF

_R1T = np.array(
    [((j << 28) | 0x0FFFFFFF) ^ 0x80000000 for j in range(16)], dtype=np.uint32
).view(np.int32)


def _sortkey(x):
    i = jax.lax.bitcast_convert_type(x, jnp.int32)
    return jnp.where(i >= 0, i, i ^ 0x7FFFFFFF)


def _softplus(x):
    return jnp.maximum(x, 0.0) + jnp.log1p(jnp.exp(-jnp.abs(x)))


def _onehot_cell(r, c):
    ri = jax.lax.broadcasted_iota(jnp.int32, (8, 128), 0)
    ci = jax.lax.broadcasted_iota(jnp.int32, (8, 128), 1)
    return (ri == r) & (ci == c)


def _cell_f32(scalar, r, c):
    return jnp.where(_onehot_cell(r, c), scalar, 0.0)


def _extract_i32(arr, r, c):
    return jnp.sum(jnp.where(_onehot_cell(r, c), arr, 0))


def _extract_f32(arr, r, c):
    return jnp.sum(jnp.where(_onehot_cell(r, c), arr, 0.0))


def _cumsum_last(x, n):
    sh = 1
    while sh < n:
        pad = jnp.zeros_like(jax.lax.slice_in_dim(x, 0, sh, axis=x.ndim - 1))
        x = x + jnp.concatenate(
            [pad, jax.lax.slice_in_dim(x, 0, x.shape[-1] - sh, axis=x.ndim - 1)],
            axis=x.ndim - 1,
        )
        sh *= 2
    return x


def _cumsum_axis(x, axis, n):
    sh = 1
    while sh < n:
        pad = jnp.zeros_like(jax.lax.slice_in_dim(x, 0, sh, axis=axis))
        x = x + jnp.concatenate(
            [pad, jax.lax.slice_in_dim(x, 0, x.shape[axis] - sh, axis=axis)],
            axis=axis,
        )
        sh *= 2
    return x


def _dist_kernel(e1_ref, e2_ref, d1_ref, d2_ref):
    eye = (
        jax.lax.broadcasted_iota(jnp.int32, (_B, _B), 0)
        == jax.lax.broadcasted_iota(jnp.int32, (_B, _B), 1)
    ).astype(jnp.float32)
    for e_ref, d_ref in ((e1_ref, d1_ref), (e2_ref, d2_ref)):
        e = e_ref[...]
        g = jax.lax.dot_general(
            e, e, (((1,), (1,)), ((), ())), preferred_element_type=jnp.float32
        )
        ncol = jnp.sum(e * e, axis=1, keepdims=True)
        nrow = jnp.sum(g * eye, axis=0, keepdims=True)
        d_ref[...] = ncol + nrow - 2.0 * g


def _valid_mask(pv_ref, nv_ref):
    pvb = pv_ref[...]
    nvb = nv_ref[...]
    return pvb[:, :, None] * nvb[:, None, :]


def _masked_keys(d_ref, vmb):
    da = d_ref[...]
    diff = da[:, :, None] - da[:, None, :]
    key = jnp.where(vmb, _sortkey(diff), _BIG)
    return diff, key


def _count_kernel(thr_ref, d1_ref, d2_ref, pv_ref, nv_ref, out_ref, *, first_round):
    a = pl.program_id(0)

    @pl.when(a == 0)
    def _init():
        out_ref[...] = jnp.zeros((8, 128), jnp.float32)

    vmf = _valid_mask(pv_ref, nv_ref)
    vmb = vmf > 0
    contrib = jnp.zeros((8, 128), jnp.float32)
    for t_idx, d_ref in ((0, d1_ref), (1, d2_ref)):
        _, key = _masked_keys(d_ref, vmb)
        if first_round and t_idx == 0:
            contrib = contrib + _cell_f32(jnp.sum(vmf), 0, 0)
        for j in range(16):
            if first_round:
                t = int(_R1T[j])
            else:
                t = _extract_i32(thr_ref[...], t_idx, j)
            cnt = jnp.sum(jnp.where(key <= t, 1.0, 0.0))
            contrib = contrib + _cell_f32(cnt, t_idx + 1, j)
    out_ref[...] = out_ref[...] + contrib


def _final_kernel(sc_ref, d1_ref, d2_ref, pv_ref, nv_ref, out_ref):
    a = pl.program_id(0)

    @pl.when(a == 0)
    def _init():
        out_ref[...] = jnp.zeros((8, 128), jnp.float32)

    sc = sc_ref[...]
    v1 = _extract_i32(sc, 0, 0)
    v2 = _extract_i32(sc, 0, 1)
    r1 = _extract_i32(sc, 0, 2).astype(jnp.float32)
    r2 = _extract_i32(sc, 0, 3).astype(jnp.float32)

    vmf = _valid_mask(pv_ref, nv_ref)
    vmb = vmf > 0
    diff1, key1 = _masked_keys(d1_ref, vmb)
    diff2, key2 = _masked_keys(d2_ref, vmb)
    loss1 = _softplus(diff1)
    loss2 = _softplus(diff2)

    old = out_ref[...]
    run2 = _extract_f32(old, 1, 0)
    run1 = _extract_f32(old, 1, 1)

    contrib = jnp.zeros((8, 128), jnp.float32)
    contrib = contrib + _cell_f32(jnp.sum(vmf * loss1), 0, 2)
    contrib = contrib + _cell_f32(jnp.sum(vmf * loss2), 0, 3)

    for (sel_key, vstar, run, rneed, other_loss, col, runcol) in (
        (key2, v2, run2, r2, loss1, 0, 0),
        (key1, v1, run1, r1, loss2, 1, 1),
    ):
        tkey = sel_key & _TRUNC
        lt = tkey < vstar
        tie = (tkey == vstar).astype(jnp.float32)
        s_lt = jnp.sum(jnp.where(lt, other_loss, 0.0))
        cs_n = _cumsum_last(tie, _B)
        row_tot = jax.lax.slice_in_dim(cs_n, _B - 1, _B, axis=2)
        incl_p = _cumsum_axis(row_tot, 1, _B)
        excl_p = incl_p - row_tot
        a_tot = jax.lax.slice_in_dim(incl_p, _B - 1, _B, axis=1)
        incl_a = _cumsum_axis(a_tot, 0, _ABLK)
        excl_a = incl_a - a_tot
        local_excl = (cs_n - tie) + excl_p + excl_a
        take = (tie > 0) & (run + local_excl < rneed)
        s_tie = jnp.sum(jnp.where(take, other_loss, 0.0))
        contrib = contrib + _cell_f32(s_lt + s_tie, 0, col)
        contrib = contrib + _cell_f32(jnp.sum(tie), 1, runcol)

    out_ref[...] = old + contrib


def _sweep_call(kern, d1, d2, pv, nv, carrier):
    return pl.pallas_call(
        kern,
        grid=(_GRID,),
        in_specs=[
            pl.BlockSpec((8, 128), lambda a: (0, 0)),
            pl.BlockSpec((_ABLK, _B), lambda a: (a, 0)),
            pl.BlockSpec((_ABLK, _B), lambda a: (a, 0)),
            pl.BlockSpec((_ABLK, _B), lambda a: (a, 0)),
            pl.BlockSpec((_ABLK, _B), lambda a: (a, 0)),
        ],
        out_specs=pl.BlockSpec((8, 128), lambda a: (0, 0)),
        out_shape=jax.ShapeDtypeStruct((8, 128), jnp.float32),
    )(carrier, d1, d2, pv, nv)


def kernel(emb1, emb2, targets, keep_rate):
    same = targets[:, None] == targets[None, :]
    idx = jnp.arange(_B)
    pv = (same & (idx[:, None] < idx[None, :])).astype(jnp.float32)
    nv = (~same).astype(jnp.float32)

    d1, d2 = pl.pallas_call(
        _dist_kernel,
        out_shape=(
            jax.ShapeDtypeStruct((_B, _B), jnp.float32),
            jax.ShapeDtypeStruct((_B, _B), jnp.float32),
        ),
    )(emb1, emb2)

    zero_c = jnp.zeros((8, 128), jnp.int32)
    o2 = _sweep_call(
        functools.partial(_count_kernel, first_round=True), d1, d2, pv, nv, zero_c
    )
    num_valid_f = o2[0, 0]
    num_valid = num_valid_f.astype(jnp.int32)
    num_keep = jnp.floor(keep_rate * num_valid).astype(jnp.int32)
    kf = num_keep.astype(jnp.float32)

    jarr = jnp.arange(16, dtype=jnp.uint32)
    bit31 = jnp.uint32(0x80000000)

    def pick(C, prefix, cbelow, shift):
        ge = C >= kf
        jstar = jnp.argmax(ge).astype(jnp.uint32)
        cb_new = jnp.where(
            jstar > 0, C[jnp.maximum(jstar.astype(jnp.int32) - 1, 0)], cbelow
        )
        return prefix | (jstar << shift), cb_new

    p1, cb1 = pick(o2[1, :16], jnp.uint32(0), jnp.float32(0.0), 28)
    p2, cb2 = pick(o2[2, :16], jnp.uint32(0), jnp.float32(0.0), 28)

    for shift in (24, 20):
        low = jnp.uint32((1 << shift) - 1)
        t1 = jax.lax.bitcast_convert_type(
            (p1 | (jarr << shift) | low) ^ bit31, jnp.int32
        )
        t2 = jax.lax.bitcast_convert_type(
            (p2 | (jarr << shift) | low) ^ bit31, jnp.int32
        )
        carrier = zero_c.at[0, :16].set(t1).at[1, :16].set(t2)
        o3 = _sweep_call(
            functools.partial(_count_kernel, first_round=False), d1, d2, pv, nv, carrier
        )
        p1, cb1 = pick(o3[1, :16], p1, cb1, shift)
        p2, cb2 = pick(o3[2, :16], p2, cb2, shift)

    v1s = jax.lax.bitcast_convert_type(p1 ^ bit31, jnp.int32)
    v2s = jax.lax.bitcast_convert_type(p2 ^ bit31, jnp.int32)
    r1 = num_keep - cb1.astype(jnp.int32)
    r2 = num_keep - cb2.astype(jnp.int32)

    sc = (
        zero_c.at[0, 0]
        .set(v1s)
        .at[0, 1]
        .set(v2s)
        .at[0, 2]
        .set(r1)
        .at[0, 3]
        .set(r2)
    )
    o4 = _sweep_call(_final_kernel, d1, d2, pv, nv, sc)

    upd1, upd2 = o4[0, 0], o4[0, 1]
    l1s, l2s = o4[0, 2], o4[0, 3]
    return (upd1 / num_keep, upd2 / num_keep, l1s / num_valid, l2s / num_valid)


# speedup vs baseline: 42.2925x; 1.0011x over previous
import functools

import jax
import jax.numpy as jnp
import numpy as np
from jax.experimental import pallas as pl
from jax.experimental.pallas import tpu as pltpu

_B = 256
_ABLK = 8
_GRID = _B // _ABLK
_TRUNC = -1048576
_BIG = 0x7FFFFFFF

_R1T = np.array(
    [((j << 28) | 0x0FFFFFFF) ^ 0x80000000 for j in range(16)], dtype=np.uint32
).view(np.int32)


def _sortkey(x):
    i = jax.lax.bitcast_convert_type(x, jnp.int32)
    return jnp.where(i >= 0, i, i ^ 0x7FFFFFFF)


def _softplus(x):
    return jnp.maximum(x, 0.0) + jnp.log1p(jnp.exp(-jnp.abs(x)))


def _onehot_cell(r, c):
    ri = jax.lax.broadcasted_iota(jnp.int32, (8, 128), 0)
    ci = jax.lax.broadcasted_iota(jnp.int32, (8, 128), 1)
    return (ri == r) & (ci == c)


def _cell_f32(scalar, r, c):
    return jnp.where(_onehot_cell(r, c), scalar, 0.0)


def _extract_i32(arr, r, c):
    return jnp.sum(jnp.where(_onehot_cell(r, c), arr, 0))


def _extract_f32(arr, r, c):
    return jnp.sum(jnp.where(_onehot_cell(r, c), arr, 0.0))


def _cumsum_last(x, n):
    sh = 1
    while sh < n:
        pad = jnp.zeros_like(jax.lax.slice_in_dim(x, 0, sh, axis=x.ndim - 1))
        x = x + jnp.concatenate(
            [pad, jax.lax.slice_in_dim(x, 0, x.shape[-1] - sh, axis=x.ndim - 1)],
            axis=x.ndim - 1,
        )
        sh *= 2
    return x


def _cumsum_axis(x, axis, n):
    sh = 1
    while sh < n:
        pad = jnp.zeros_like(jax.lax.slice_in_dim(x, 0, sh, axis=axis))
        x = x + jnp.concatenate(
            [pad, jax.lax.slice_in_dim(x, 0, x.shape[axis] - sh, axis=axis)],
            axis=axis,
        )
        sh *= 2
    return x


def _dist_kernel(e1_ref, e2_ref, d1_ref, d2_ref):
    eye = (
        jax.lax.broadcasted_iota(jnp.int32, (_B, _B), 0)
        == jax.lax.broadcasted_iota(jnp.int32, (_B, _B), 1)
    ).astype(jnp.float32)
    for e_ref, d_ref in ((e1_ref, d1_ref), (e2_ref, d2_ref)):
        e = e_ref[...]
        g = jax.lax.dot_general(
            e, e, (((1,), (1,)), ((), ())), preferred_element_type=jnp.float32
        )
        ncol = jnp.sum(e * e, axis=1, keepdims=True)
        nrow = jnp.sum(g * eye, axis=0, keepdims=True)
        d_ref[...] = ncol + nrow - 2.0 * g


def _valid_mask(pv_ref, nv_ref):
    pvb = pv_ref[...]
    nvb = nv_ref[...]
    return pvb[:, :, None] * nvb[:, None, :]


def _masked_keys(d_ref, vmb):
    da = d_ref[...]
    diff = da[:, :, None] - da[:, None, :]
    key = jnp.where(vmb, _sortkey(diff), _BIG)
    return diff, key


def _count_kernel(thr_ref, d1_ref, d2_ref, pv_ref, nv_ref, out_ref, *, first_round):
    vmf = _valid_mask(pv_ref, nv_ref)
    vmb = vmf > 0
    contrib = jnp.zeros((8, 128), jnp.float32)
    for t_idx, d_ref in ((0, d1_ref), (1, d2_ref)):
        _, key = _masked_keys(d_ref, vmb)
        if first_round and t_idx == 0:
            contrib = contrib + _cell_f32(jnp.sum(vmf), 0, 0)
        for j in range(16):
            if first_round:
                t = int(_R1T[j])
            else:
                t = _extract_i32(thr_ref[...], t_idx, j)
            cnt = jnp.sum(jnp.where(key <= t, 1.0, 0.0))
            contrib = contrib + _cell_f32(cnt, t_idx + 1, j)
    out_ref[...] = contrib[None]


def _final_kernel(sc_ref, d1_ref, d2_ref, pv_ref, nv_ref, out_ref):
    a = pl.program_id(0)

    @pl.when(a == 0)
    def _init():
        out_ref[...] = jnp.zeros((8, 128), jnp.float32)

    sc = sc_ref[...]
    v1 = _extract_i32(sc, 0, 0)
    v2 = _extract_i32(sc, 0, 1)
    r1 = _extract_i32(sc, 0, 2).astype(jnp.float32)
    r2 = _extract_i32(sc, 0, 3).astype(jnp.float32)

    vmf = _valid_mask(pv_ref, nv_ref)
    vmb = vmf > 0
    diff1, key1 = _masked_keys(d1_ref, vmb)
    diff2, key2 = _masked_keys(d2_ref, vmb)
    loss1 = _softplus(diff1)
    loss2 = _softplus(diff2)

    old = out_ref[...]
    run2 = _extract_f32(old, 1, 0)
    run1 = _extract_f32(old, 1, 1)

    contrib = jnp.zeros((8, 128), jnp.float32)
    contrib = contrib + _cell_f32(jnp.sum(vmf * loss1), 0, 2)
    contrib = contrib + _cell_f32(jnp.sum(vmf * loss2), 0, 3)

    for (sel_key, vstar, run, rneed, other_loss, col, runcol) in (
        (key2, v2, run2, r2, loss1, 0, 0),
        (key1, v1, run1, r1, loss2, 1, 1),
    ):
        tkey = sel_key & _TRUNC
        lt = tkey < vstar
        tie = (tkey == vstar).astype(jnp.float32)
        s_lt = jnp.sum(jnp.where(lt, other_loss, 0.0))
        cs_n = _cumsum_last(tie, _B)
        row_tot = jax.lax.slice_in_dim(cs_n, _B - 1, _B, axis=2)
        incl_p = _cumsum_axis(row_tot, 1, _B)
        excl_p = incl_p - row_tot
        a_tot = jax.lax.slice_in_dim(incl_p, _B - 1, _B, axis=1)
        incl_a = _cumsum_axis(a_tot, 0, _ABLK)
        excl_a = incl_a - a_tot
        local_excl = (cs_n - tie) + excl_p + excl_a
        take = (tie > 0) & (run + local_excl < rneed)
        s_tie = jnp.sum(jnp.where(take, other_loss, 0.0))
        contrib = contrib + _cell_f32(s_lt + s_tie, 0, col)
        contrib = contrib + _cell_f32(jnp.sum(tie), 1, runcol)

    out_ref[...] = old + contrib


_IN_SPECS = [
    pl.BlockSpec((8, 128), lambda a: (0, 0)),
    pl.BlockSpec((_ABLK, _B), lambda a: (a, 0)),
    pl.BlockSpec((_ABLK, _B), lambda a: (a, 0)),
    pl.BlockSpec((_ABLK, _B), lambda a: (a, 0)),
    pl.BlockSpec((_ABLK, _B), lambda a: (a, 0)),
]


def _sweep_call(kern, d1, d2, pv, nv, carrier):
    return pl.pallas_call(
        kern,
        grid=(_GRID,),
        in_specs=_IN_SPECS,
        out_specs=pl.BlockSpec((8, 128), lambda a: (0, 0)),
        out_shape=jax.ShapeDtypeStruct((8, 128), jnp.float32),
    )(carrier, d1, d2, pv, nv)


def _count_call(kern, d1, d2, pv, nv, carrier):
    partials = pl.pallas_call(
        kern,
        grid=(_GRID,),
        in_specs=_IN_SPECS,
        out_specs=pl.BlockSpec((1, 8, 128), lambda a: (a, 0, 0)),
        out_shape=jax.ShapeDtypeStruct((_GRID, 8, 128), jnp.float32),
        compiler_params=pltpu.CompilerParams(dimension_semantics=("parallel",)),
    )(carrier, d1, d2, pv, nv)
    return jnp.sum(partials, axis=0)


def kernel(emb1, emb2, targets, keep_rate):
    same = targets[:, None] == targets[None, :]
    idx = jnp.arange(_B)
    pv = (same & (idx[:, None] < idx[None, :])).astype(jnp.float32)
    nv = (~same).astype(jnp.float32)

    d1, d2 = pl.pallas_call(
        _dist_kernel,
        out_shape=(
            jax.ShapeDtypeStruct((_B, _B), jnp.float32),
            jax.ShapeDtypeStruct((_B, _B), jnp.float32),
        ),
    )(emb1, emb2)

    zero_c = jnp.zeros((8, 128), jnp.int32)
    o2 = _count_call(
        functools.partial(_count_kernel, first_round=True), d1, d2, pv, nv, zero_c
    )
    num_valid_f = o2[0, 0]
    num_valid = num_valid_f.astype(jnp.int32)
    num_keep = jnp.floor(keep_rate * num_valid).astype(jnp.int32)
    kf = num_keep.astype(jnp.float32)

    jarr = jnp.arange(16, dtype=jnp.uint32)
    bit31 = jnp.uint32(0x80000000)

    def pick(C, prefix, cbelow, shift):
        ge = C >= kf
        jstar = jnp.argmax(ge).astype(jnp.uint32)
        cb_new = jnp.where(
            jstar > 0, C[jnp.maximum(jstar.astype(jnp.int32) - 1, 0)], cbelow
        )
        return prefix | (jstar << shift), cb_new

    p1, cb1 = pick(o2[1, :16], jnp.uint32(0), jnp.float32(0.0), 28)
    p2, cb2 = pick(o2[2, :16], jnp.uint32(0), jnp.float32(0.0), 28)

    for shift in (24, 20):
        low = jnp.uint32((1 << shift) - 1)
        t1 = jax.lax.bitcast_convert_type(
            (p1 | (jarr << shift) | low) ^ bit31, jnp.int32
        )
        t2 = jax.lax.bitcast_convert_type(
            (p2 | (jarr << shift) | low) ^ bit31, jnp.int32
        )
        carrier = zero_c.at[0, :16].set(t1).at[1, :16].set(t2)
        o3 = _count_call(
            functools.partial(_count_kernel, first_round=False), d1, d2, pv, nv, carrier
        )
        p1, cb1 = pick(o3[1, :16], p1, cb1, shift)
        p2, cb2 = pick(o3[2, :16], p2, cb2, shift)

    v1s = jax.lax.bitcast_convert_type(p1 ^ bit31, jnp.int32)
    v2s = jax.lax.bitcast_convert_type(p2 ^ bit31, jnp.int32)
    r1 = num_keep - cb1.astype(jnp.int32)
    r2 = num_keep - cb2.astype(jnp.int32)

    sc = (
        zero_c.at[0, 0]
        .set(v1s)
        .at[0, 1]
        .set(v2s)
        .at[0, 2]
        .set(r1)
        .at[0, 3]
        .set(r2)
    )
    o4 = _sweep_call(_final_kernel, d1, d2, pv, nv, sc)

    upd1, upd2 = o4[0, 0], o4[0, 1]
    l1s, l2s = o4[0, 2], o4[0, 3]
    return (upd1 / num_keep, upd2 / num_keep, l1s / num_valid, l2s / num_valid)
